# baseline (speedup 1.0000x reference)
import functools

import jax
import jax.numpy as jnp
from jax import lax
from jax.experimental import pallas as pl
from jax.experimental.pallas import tpu as pltpu
from jax.experimental.pallas import tpu_sc as plsc

_NC = 2
_NS = 16
_NW = _NC * _NS
_LANES = 16


def kernel(feats, weights, table):
    B, H = feats.shape
    V, D = table.shape
    CB = 2
    RB = B // _NW
    NCHUNK = RB // CB
    NBUF = 3
    HP = 56
    RPC = CB * HP
    W128 = 4 * D

    feats_p = jnp.pad(feats, ((0, 0), (0, HP - H))).reshape(B // CB, RPC)
    feats_p = feats_p.astype(jnp.int32)
    weights_p = jnp.pad(weights, ((0, 0), (0, HP - H))).reshape(B // CB, RPC)
    table_w = table.reshape(V // 4, W128)

    mesh = plsc.VectorSubcoreMesh(core_axis_name="c", subcore_axis_name="s")

    @functools.partial(
        pl.kernel,
        out_type=jax.ShapeDtypeStruct((B, D), jnp.float32),
        mesh=mesh,
        scratch_types=[
            pltpu.VMEM((NCHUNK, RPC), jnp.int32),
            pltpu.VMEM((NCHUNK, RPC), jnp.float32),
            pltpu.VMEM((NBUF, RPC), jnp.int32),
            pltpu.VMEM((NBUF, RPC, W128), jnp.float32),
            pltpu.VMEM((RB, D), jnp.float32),
            pltpu.SemaphoreType.DMA,
        ],
        compiler_params=pltpu.CompilerParams(use_tc_tiling_on_sc=False),
    )
    def run(feats_hbm, w_hbm, table_hbm, out_hbm, idx_v, w_v, gix_v, rows_v, out_v, sem):
        wid = lax.axis_index("s") * _NC + lax.axis_index("c")
        chunk0 = wid * NCHUNK

        pltpu.sync_copy(feats_hbm.at[pl.ds(chunk0, NCHUNK)], idx_v)
        pltpu.sync_copy(w_hbm.at[pl.ds(chunk0, NCHUNK)], w_v)

        def fire(g, b):
            for o in range(0, RPC, _LANES):
                gix_v[b, pl.ds(o, _LANES)] = lax.shift_right_logical(
                    idx_v[g, pl.ds(o, _LANES)], 2)
            pltpu.async_copy(table_hbm.at[gix_v.at[b]], rows_v.at[b], sem)

        def wait(b):
            pltpu.make_async_copy(
                table_hbm.at[gix_v.at[b]], rows_v.at[b], sem).wait()

        def compute(g, b):
            for cb in range(CB):
                base = cb * HP
                wvecs = [w_v[g, pl.ds(base + o, _LANES)] for o in (0, 16, 32, 40)]
                svecs = [
                    (idx_v[g, pl.ds(base + o, _LANES)] & 3) * D
                    for o in (0, 16, 32, 40)
                ]
                acc0 = jnp.zeros((_LANES,), jnp.float32)
                acc1 = jnp.zeros((_LANES,), jnp.float32)
                for l in range(H):
                    r = base + l
                    j, k = (l // 16, l % 16) if l < 48 else (3, l - 40)
                    w = wvecs[j][k]
                    soff = svecs[j][k]
                    acc0 = acc0 + w * rows_v[b, r, pl.ds(soff, _LANES)]
                    acc1 = acc1 + w * rows_v[b, r, pl.ds(soff + _LANES, _LANES)]
                out_v[g * CB + cb, pl.ds(0, _LANES)] = acc0
                out_v[g * CB + cb, pl.ds(_LANES, _LANES)] = acc1

        for b in range(NBUF):
            fire(b, b)

        @pl.loop(0, NCHUNK - 4, step=NBUF)
        def _(g0):
            for b in range(NBUF):
                g = g0 + b
                wait(b)
                compute(g, b)
                fire(g + NBUF, b)

        for b in range(NBUF):
            g = NCHUNK - 4 + b
            wait(b)
            compute(g, b)

        fire(NCHUNK - 1, 0)
        wait(0)
        compute(NCHUNK - 1, 0)

        pltpu.sync_copy(out_v, out_hbm.at[pl.ds(wid * RB, RB)])

    return run(feats_p, weights_p, table_w)
